# Initial kernel scaffold; baseline (speedup 1.0000x reference)
import jax
import jax.numpy as jnp
from jax import lax
from jax.experimental import pallas as pl
from jax.experimental.pallas import tpu as pltpu
from jax.experimental.pallas import tpu_sc as plsc

N = 10000
E = 320000
D = 128

NC = 2
NS = 16
NW = NC * NS
EPT = E // NW
B = 80
ROWS_PER_TILE = EPT // B
NROW = E // B
SEG = N // NS
SEG_CHUNK = 125


def _tc_project_body(x_ref, w_ref, att2_ref, h_ref, al_ref):
    h = jnp.dot(x_ref[...], w_ref[...])
    h_ref[...] = h
    al_ref[...] = lax.dot_general(att2_ref[...], h, (((1,), (1,)), ((), ())))


def _tc_project(x, W, att2):
    return pl.pallas_call(
        _tc_project_body,
        out_shape=[
            jax.ShapeDtypeStruct((N, D), jnp.float32),
            jax.ShapeDtypeStruct((2, N), jnp.float32),
        ],
    )(x, W, att2)


def _sc_denom_body(src_hbm, dst_hbm, al_hbm, dpart_hbm,
                   src_v, dst_v, as_v, ad_v, den_v, sem):
    c = lax.axis_index("c")
    s = lax.axis_index("s")
    wid = s * NC + c
    r0 = wid * ROWS_PER_TILE

    pltpu.async_copy(src_hbm.at[pl.ds(r0, ROWS_PER_TILE)], src_v, sem).wait()
    pltpu.async_copy(dst_hbm.at[pl.ds(r0, ROWS_PER_TILE)], dst_v, sem).wait()
    pltpu.async_copy(al_hbm.at[0], as_v, sem).wait()
    pltpu.async_copy(al_hbm.at[1], ad_v, sem).wait()

    @pl.loop(0, N // 16)
    def _zero(i):
        den_v[pl.ds(i * 16, 16)] = jnp.zeros((16,), jnp.float32)

    @pl.loop(0, ROWS_PER_TILE)
    def _row(j):
        for i in range(B // 16):
            s16 = src_v[j, pl.ds(i * 16, 16)]
            d16 = dst_v[j, pl.ds(i * 16, 16)]
            a = plsc.load_gather(as_v, [s16])
            b = plsc.load_gather(ad_v, [d16])
            e = a + b
            w = jnp.exp(jnp.maximum(e, 0.2 * e))
            plsc.addupdate_scatter(den_v, [d16], w)

    pltpu.async_copy(den_v, dpart_hbm.at[wid], sem).wait()


def _sc_denom(src2, dst2, alphas):
    mesh = plsc.VectorSubcoreMesh(core_axis_name="c", subcore_axis_name="s")
    return pl.kernel(
        _sc_denom_body,
        out_type=jax.ShapeDtypeStruct((NW, N), jnp.float32),
        mesh=mesh,
        scratch_types=[
            pltpu.VMEM((ROWS_PER_TILE, B), jnp.int32),
            pltpu.VMEM((ROWS_PER_TILE, B), jnp.int32),
            pltpu.VMEM((N,), jnp.float32),
            pltpu.VMEM((N,), jnp.float32),
            pltpu.VMEM((N,), jnp.float32),
            pltpu.SemaphoreType.DMA,
        ],
    )(src2, dst2, alphas)


def _tc_dinv_body(dpart_ref, dinv_ref):
    denom = jnp.sum(dpart_ref[...], axis=0, keepdims=True)
    dinv_ref[...] = 1.0 / (denom + 1e-16)


def _tc_dinv(dpart):
    return pl.pallas_call(
        _tc_dinv_body,
        out_shape=jax.ShapeDtypeStruct((1, N), jnp.float32),
    )(dpart)


def _sc_message_body(h_hbm, src_hbm, dst_hbm, al_hbm, dinv_hbm, out_hbm,
                     src_v, dst_v, as_v, ad_v, di_v, coef_v, rows_v, zbuf,
                     out_sh, sem):
    c = lax.axis_index("c")
    s = lax.axis_index("s")
    wid = s * NC + c
    r0 = wid * ROWS_PER_TILE

    pltpu.async_copy(src_hbm.at[pl.ds(r0, ROWS_PER_TILE)], src_v, sem).wait()
    pltpu.async_copy(dst_hbm.at[pl.ds(r0, ROWS_PER_TILE)], dst_v, sem).wait()
    pltpu.async_copy(al_hbm.at[0], as_v, sem).wait()
    pltpu.async_copy(al_hbm.at[1], ad_v, sem).wait()
    pltpu.async_copy(dinv_hbm.at[0], di_v, sem).wait()

    @pl.loop(0, ROWS_PER_TILE)
    def _coef(j):
        for i in range(B // 16):
            s16 = src_v[j, pl.ds(i * 16, 16)]
            d16 = dst_v[j, pl.ds(i * 16, 16)]
            a = plsc.load_gather(as_v, [s16])
            b = plsc.load_gather(ad_v, [d16])
            e = a + b
            w = jnp.exp(jnp.maximum(e, 0.2 * e))
            di = plsc.load_gather(di_v, [d16])
            coef_v[pl.ds(j * B + i * 16, 16)] = w * di

    @pl.loop(0, SEG_CHUNK)
    def _z(i):
        for cc in range(D // 16):
            zbuf[i, pl.ds(cc * 16, 16)] = jnp.zeros((16,), jnp.float32)

    for q in range(SEG // SEG_CHUNK):
        pltpu.async_copy(
            zbuf, out_sh.at[pl.ds(s * SEG + q * SEG_CHUNK, SEG_CHUNK)], sem
        ).wait()
    plsc.subcore_barrier()

    @pl.loop(0, ROWS_PER_TILE)
    def _chunk(j):
        pltpu.async_copy(h_hbm.at[src_v.at[j]], rows_v, sem).wait()

        @pl.loop(0, B)
        def _scale(r):
            cvec = plsc.load_gather(
                coef_v, [jnp.full((16,), j * B + r, jnp.int32)]
            )
            for cc in range(D // 16):
                sl = pl.ds(cc * 16, 16)
                rows_v[r, sl] = rows_v[r, sl] * cvec

        pltpu.async_copy(rows_v, out_sh.at[dst_v.at[j]], sem, add=True).wait()

    plsc.subcore_barrier()

    for q in range(SEG // SEG_CHUNK):
        sl = pl.ds(s * SEG + q * SEG_CHUNK, SEG_CHUNK)
        pltpu.async_copy(out_sh.at[sl], out_hbm.at[c].at[sl], sem).wait()


def _sc_message(h, src2, dst2, alphas, dinv):
    mesh = plsc.VectorSubcoreMesh(core_axis_name="c", subcore_axis_name="s")
    return pl.kernel(
        _sc_message_body,
        out_type=jax.ShapeDtypeStruct((NC, N, D), jnp.float32),
        mesh=mesh,
        scratch_types=[
            pltpu.VMEM((ROWS_PER_TILE, B), jnp.int32),
            pltpu.VMEM((ROWS_PER_TILE, B), jnp.int32),
            pltpu.VMEM((N,), jnp.float32),
            pltpu.VMEM((N,), jnp.float32),
            pltpu.VMEM((N,), jnp.float32),
            pltpu.VMEM((EPT,), jnp.float32),
            pltpu.VMEM((B, D), jnp.float32),
            pltpu.VMEM((SEG_CHUNK, D), jnp.float32),
            pltpu.VMEM_SHARED((N, D), jnp.float32),
            pltpu.SemaphoreType.DMA,
        ],
    )(h, src2, dst2, alphas, dinv)


def _tc_finish_body(p_ref, b_ref, o_ref):
    o_ref[...] = jnp.maximum(p_ref[0] + p_ref[1] + b_ref[...], 0.0)


def _tc_finish(parts, bias2):
    return pl.pallas_call(
        _tc_finish_body,
        out_shape=jax.ShapeDtypeStruct((N, D), jnp.float32),
    )(parts, bias2)


@jax.jit
def kernel(x, edge_index, W, att_src, att_dst, bias):
    att2 = jnp.stack([att_src, att_dst], axis=0)
    h, alphas = _tc_project(x, W, att2)
    src2 = edge_index[0].reshape(NROW, B)
    dst2 = edge_index[1].reshape(NROW, B)
    dpart = _sc_denom(src2, dst2, alphas)
    dinv = _tc_dinv(dpart)
    parts = _sc_message(h, src2, dst2, alphas, dinv)
    return _tc_finish(parts, bias.reshape(1, D))


# speedup vs baseline: 21.9561x; 21.9561x over previous
import dataclasses

import jax
import jax.numpy as jnp
from jax import lax
from jax.experimental import pallas as pl
from jax.experimental.pallas import tpu as pltpu
from jax.experimental.pallas import tpu_sc as plsc

N = 10000
E = 320000
D = 128

NC = 2
NS = 16
NW = NC * NS
EPT = E // NW
B = 80
ROWS_PER_TILE = EPT // B
NSEG = N // B


def _sc_compiler_params():
    cp = pltpu.CompilerParams()
    if "needs_layout_passes" in pltpu.CompilerParams.__dataclass_fields__:
        cp = dataclasses.replace(cp, needs_layout_passes=False)
    return cp


def _tc_project_body(x_ref, w_ref, att2_ref, h_ref, al_ref):
    h = jnp.dot(x_ref[...], w_ref[...])
    h_ref[...] = h
    al_ref[...] = lax.dot_general(att2_ref[...], h, (((1,), (1,)), ((), ())))


def _tc_project(x, W, att2):
    return pl.pallas_call(
        _tc_project_body,
        out_shape=[
            jax.ShapeDtypeStruct((N, D), jnp.float32),
            jax.ShapeDtypeStruct((2, N), jnp.float32),
        ],
    )(x, W, att2)


def _sc_denom_body(src_hbm, dst_hbm, as_hbm, ad_hbm, dpart_hbm, w_hbm,
                   src_v, dst_v, as_v, ad_v, den_v, w_v, sem):
    c = lax.axis_index("c")
    s = lax.axis_index("s")
    wid = s * NC + c

    pltpu.async_copy(src_hbm.at[wid], src_v, sem).wait()
    pltpu.async_copy(dst_hbm.at[wid], dst_v, sem).wait()
    pltpu.async_copy(as_hbm, as_v, sem).wait()
    pltpu.async_copy(ad_hbm, ad_v, sem).wait()

    @pl.loop(0, N // 16)
    def _zero(i):
        den_v[pl.ds(i * 16, 16)] = jnp.zeros((16,), jnp.float32)

    @pl.loop(0, ROWS_PER_TILE)
    def _row(j):
        for i in range(B // 16):
            s16 = src_v[j, pl.ds(i * 16, 16)]
            d16 = dst_v[j, pl.ds(i * 16, 16)]
            a = plsc.load_gather(as_v, [s16])
            b = plsc.load_gather(ad_v, [d16])
            e = a + b
            w = jnp.exp(jnp.maximum(e, 0.2 * e))
            w_v[j, pl.ds(i * 16, 16)] = w
            plsc.addupdate_scatter(den_v, [d16], w)

    pltpu.async_copy(den_v, dpart_hbm.at[pl.ds(wid * N, N)], sem).wait()
    pltpu.async_copy(w_v, w_hbm.at[wid], sem).wait()


def _sc_denom(src3, dst3, as_h, ad_h):
    mesh = plsc.VectorSubcoreMesh(core_axis_name="c", subcore_axis_name="s")
    return pl.kernel(
        _sc_denom_body,
        out_type=[
            jax.ShapeDtypeStruct((NW * N,), jnp.float32),
            jax.ShapeDtypeStruct((NW, ROWS_PER_TILE, B), jnp.float32),
        ],
        mesh=mesh,
        scratch_types=[
            pltpu.VMEM((ROWS_PER_TILE, B), jnp.int32),
            pltpu.VMEM((ROWS_PER_TILE, B), jnp.int32),
            pltpu.VMEM((N,), jnp.float32),
            pltpu.VMEM((N,), jnp.float32),
            pltpu.VMEM((N,), jnp.float32),
            pltpu.VMEM((ROWS_PER_TILE, B), jnp.float32),
            pltpu.SemaphoreType.DMA,
        ],
        compiler_params=_sc_compiler_params(),
    )(src3, dst3, as_h, ad_h)


def _tc_dinv_body(dpart_ref, dinv_ref):
    denom = jnp.sum(dpart_ref[...], axis=0, keepdims=True)
    dinv_ref[...] = 1.0 / (denom + 1e-16)


def _tc_dinv(dpart):
    return pl.pallas_call(
        _tc_dinv_body,
        out_shape=jax.ShapeDtypeStruct((1, N), jnp.float32),
    )(dpart)


def _sc_message_body(h_hbm, src_hbm, dst_hbm, w_hbm, dinv_hbm, out_hbm,
                     sbuf, dst_v, wbuf, cbuf, di_v, rows_v, zbuf, out_sh,
                     sem):
    c = lax.axis_index("c")
    s = lax.axis_index("s")
    wid = s * NC + c

    pltpu.async_copy(dst_hbm.at[wid], dst_v, sem).wait()
    pltpu.async_copy(dinv_hbm, di_v, sem).wait()

    @pl.loop(0, 8)
    def _z(i):
        for cc in range(D // 16):
            zbuf[i, pl.ds(cc * 16, 16)] = jnp.zeros((16,), jnp.float32)

    for k in range((NSEG + NS - 1) // NS):
        j = s + k * NS

        @pl.when(j < NSEG)
        def _():
            for q in range(B // 8):
                pltpu.async_copy(
                    zbuf, out_sh.at[pl.ds(j * B + q * 8, 8)], sem
                ).wait()

    plsc.subcore_barrier()

    @pl.loop(0, ROWS_PER_TILE)
    def _chunk(j):
        pltpu.async_copy(src_hbm.at[wid].at[j], sbuf, sem).wait()
        pltpu.async_copy(w_hbm.at[wid].at[j], wbuf, sem).wait()

        for i in range(B // 16):
            d16 = dst_v[j, pl.ds(i * 16, 16)]
            di = plsc.load_gather(di_v, [d16])
            cbuf[pl.ds(i * 16, 16)] = wbuf[pl.ds(i * 16, 16)] * di

        pltpu.async_copy(h_hbm.at[sbuf], rows_v, sem).wait()

        @pl.loop(0, B)
        def _scale(r):
            cvec = plsc.load_gather(cbuf, [jnp.full((16,), r, jnp.int32)])
            for cc in range(D // 16):
                sl = pl.ds(cc * 16, 16)
                rows_v[r, sl] = rows_v[r, sl] * cvec

        pltpu.async_copy(rows_v, out_sh.at[dst_v.at[j]], sem, add=True).wait()

    plsc.subcore_barrier()

    for k in range((NSEG + NS - 1) // NS):
        j = s + k * NS

        @pl.when(j < NSEG)
        def _():
            sl = pl.ds(j * B, B)
            pltpu.async_copy(out_sh.at[sl], out_hbm.at[c].at[sl], sem).wait()


def _sc_message(h, src3, dst3, w3, dinv):
    mesh = plsc.VectorSubcoreMesh(core_axis_name="c", subcore_axis_name="s")
    return pl.kernel(
        _sc_message_body,
        out_type=jax.ShapeDtypeStruct((NC, N, D), jnp.float32),
        mesh=mesh,
        scratch_types=[
            pltpu.VMEM((B,), jnp.int32),
            pltpu.VMEM((ROWS_PER_TILE, B), jnp.int32),
            pltpu.VMEM((B,), jnp.float32),
            pltpu.VMEM((B,), jnp.float32),
            pltpu.VMEM((N,), jnp.float32),
            pltpu.VMEM((B, D), jnp.float32),
            pltpu.VMEM((8, D), jnp.float32),
            pltpu.VMEM_SHARED((N, D), jnp.float32),
            pltpu.SemaphoreType.DMA,
        ],
        compiler_params=_sc_compiler_params(),
    )(h, src3, dst3, w3, dinv)


def _tc_finish_body(p_ref, b_ref, o_ref):
    o_ref[...] = jnp.maximum(p_ref[0] + p_ref[1] + b_ref[...], 0.0)


def _tc_finish(parts, bias2):
    return pl.pallas_call(
        _tc_finish_body,
        out_shape=jax.ShapeDtypeStruct((N, D), jnp.float32),
    )(parts, bias2)


@jax.jit
def kernel(x, edge_index, W, att_src, att_dst, bias):
    att2 = jnp.stack([att_src, att_dst], axis=0)
    h, alphas = _tc_project(x, W, att2)
    as_h = alphas[0]
    ad_h = alphas[1]
    src3 = edge_index[0].reshape(NW, ROWS_PER_TILE, B)
    dst3 = edge_index[1].reshape(NW, ROWS_PER_TILE, B)
    dpart, w3 = _sc_denom(src3, dst3, as_h, ad_h)
    dinv = _tc_dinv(dpart.reshape(NW, N)).reshape(N)
    parts = _sc_message(h, src3, dst3, w3, dinv)
    return _tc_finish(parts, bias.reshape(1, D))


# speedup vs baseline: 41.6921x; 1.8989x over previous
import dataclasses

import jax
import jax.numpy as jnp
from jax import lax
from jax.experimental import pallas as pl
from jax.experimental.pallas import tpu as pltpu
from jax.experimental.pallas import tpu_sc as plsc

N = 10000
E = 320000
D = 128

NC = 2
NS = 16
NW = NC * NS
EPT = E // NW
B = 80
ROWS_PER_TILE = EPT // B
NSEG = N // B
ZK = (NSEG + NS - 1) // NS


def _sc_compiler_params():
    cp = pltpu.CompilerParams()
    if "needs_layout_passes" in pltpu.CompilerParams.__dataclass_fields__:
        cp = dataclasses.replace(cp, needs_layout_passes=False)
    return cp


def _tc_alphas_body(x_ref, w_ref, att2_ref, al_ref):
    av2 = lax.dot_general(w_ref[...], att2_ref[...], (((1,), (1,)), ((), ())))
    al_ref[...] = lax.dot_general(av2, x_ref[...], (((0,), (1,)), ((), ())))


def _tc_alphas(x, W, att2):
    return pl.pallas_call(
        _tc_alphas_body,
        out_shape=jax.ShapeDtypeStruct((2, N), jnp.float32),
    )(x, W, att2)


def _tc_h_body(x_ref, w_ref, h_ref):
    h_ref[...] = jnp.dot(x_ref[...], w_ref[...])


def _tc_h(x, W):
    return pl.pallas_call(
        _tc_h_body,
        out_shape=jax.ShapeDtypeStruct((N, D), jnp.float32),
    )(x, W)


def _sc_denom_body(src_hbm, dst_hbm, as_hbm, ad_hbm, dpart_hbm, w_hbm,
                   src_v, dst_v, as_v, ad_v, den_v, w_v, sem):
    c = lax.axis_index("c")
    s = lax.axis_index("s")
    wid = s * NC + c

    pltpu.async_copy(src_hbm.at[wid], src_v, sem).wait()
    pltpu.async_copy(dst_hbm.at[wid], dst_v, sem).wait()
    pltpu.async_copy(as_hbm, as_v, sem).wait()
    pltpu.async_copy(ad_hbm, ad_v, sem).wait()

    @pl.loop(0, N // 16)
    def _zero(i):
        den_v[pl.ds(i * 16, 16)] = jnp.zeros((16,), jnp.float32)

    @pl.loop(0, ROWS_PER_TILE)
    def _row(j):
        for i in range(B // 16):
            s16 = src_v[j, pl.ds(i * 16, 16)]
            d16 = dst_v[j, pl.ds(i * 16, 16)]
            a = plsc.load_gather(as_v, [s16])
            b = plsc.load_gather(ad_v, [d16])
            e = a + b
            w = jnp.exp(jnp.maximum(e, 0.2 * e))
            w_v[j, pl.ds(i * 16, 16)] = w
            plsc.addupdate_scatter(den_v, [d16], w)

    pltpu.async_copy(den_v, dpart_hbm.at[pl.ds(wid * N, N)], sem).wait()
    pltpu.async_copy(w_v, w_hbm.at[wid], sem).wait()


def _sc_denom(src3, dst3, as_h, ad_h):
    mesh = plsc.VectorSubcoreMesh(core_axis_name="c", subcore_axis_name="s")
    return pl.kernel(
        _sc_denom_body,
        out_type=[
            jax.ShapeDtypeStruct((NW * N,), jnp.float32),
            jax.ShapeDtypeStruct((NW, ROWS_PER_TILE, B), jnp.float32),
        ],
        mesh=mesh,
        scratch_types=[
            pltpu.VMEM((ROWS_PER_TILE, B), jnp.int32),
            pltpu.VMEM((ROWS_PER_TILE, B), jnp.int32),
            pltpu.VMEM((N,), jnp.float32),
            pltpu.VMEM((N,), jnp.float32),
            pltpu.VMEM((N,), jnp.float32),
            pltpu.VMEM((ROWS_PER_TILE, B), jnp.float32),
            pltpu.SemaphoreType.DMA,
        ],
        compiler_params=_sc_compiler_params(),
    )(src3, dst3, as_h, ad_h)


def _tc_dinv_body(dpart_ref, dinv_ref):
    denom = jnp.sum(dpart_ref[...], axis=0, keepdims=True)
    dinv_ref[...] = 1.0 / (denom + 1e-16)


def _tc_dinv(dpart):
    return pl.pallas_call(
        _tc_dinv_body,
        out_shape=jax.ShapeDtypeStruct((1, N), jnp.float32),
    )(dpart)


UNROLL = 5
NBODY = ROWS_PER_TILE // UNROLL
NRB = 3


def _sc_message_body(h_hbm, src_hbm, dst_hbm, w_hbm, dinv_hbm, out_hbm,
                     src5, dst5, dsc5, w5, c5, di_v, rows0, rows1, rows2,
                     sem_i0, sem_i1, sem_i2, sem_i3, sem_i4,
                     sem_g0, sem_g1, sem_g2, sem_s0, sem_s1, sem_s2, sem_z,
                     out_sh):
    c = lax.axis_index("c")
    s = lax.axis_index("s")
    wid = s * NC + c

    rows = (rows0, rows1, rows2)
    sem_i = (sem_i0, sem_i1, sem_i2, sem_i3, sem_i4)
    sem_g = (sem_g0, sem_g1, sem_g2)
    sem_s = (sem_s0, sem_s1, sem_s2)

    pltpu.async_copy(dinv_hbm, di_v, sem_z).wait()

    @pl.loop(0, B)
    def _zr(r):
        for cc in range(D // 16):
            rows0[r, pl.ds(cc * 16, 16)] = jnp.zeros((16,), jnp.float32)

    zh = []
    for k in range(ZK):
        j = jnp.minimum(s + k * NS, NSEG - 1)
        zh.append(
            pltpu.async_copy(rows0, out_sh.at[pl.ds(j * B, B)], sem_z))
    for hnd in zh:
        hnd.wait()

    plsc.subcore_barrier()

    def coef(i):
        for g in range(B // 16):
            sl = pl.ds(g * 16, 16)
            d16 = dst5[i, sl]
            di = plsc.load_gather(di_v, [d16])
            c5[pl.ds(i * B + g * 16, 16)] = w5[i, sl] * di
            dsc5[i, sl] = d16

    def scale(i):
        @pl.loop(0, B, step=2)
        def _scale(r):
            for rr in range(2):
                cvec = plsc.load_gather(
                    c5, [jnp.full((16,), i * B + r + rr, jnp.int32)]
                )
                for cc in range(D // 16):
                    sl = pl.ds(cc * 16, 16)
                    rows[i % NRB][r + rr, sl] = (
                        rows[i % NRB][r + rr, sl] * cvec)

    @pl.loop(0, NBODY)
    def _outer(t):
        base = t * UNROLL
        ih = []
        for i in range(UNROLL):
            ih.append([
                pltpu.async_copy(src_hbm.at[wid].at[base + i],
                                 src5.at[i], sem_i[i]),
                pltpu.async_copy(dst_hbm.at[wid].at[base + i],
                                 dst5.at[i], sem_i[i]),
                pltpu.async_copy(w_hbm.at[wid].at[base + i],
                                 w5.at[i], sem_i[i]),
            ])

        gh = [None] * UNROLL
        sh = [None] * UNROLL
        for hnd in ih[0]:
            hnd.wait()
        gh[0] = pltpu.async_copy(h_hbm.at[src5.at[0]], rows[0], sem_g[0])
        for hnd in ih[1]:
            hnd.wait()
        gh[1] = pltpu.async_copy(h_hbm.at[src5.at[1]], rows[1], sem_g[1])
        coef(0)
        coef(1)

        for i in range(UNROLL):
            gh[i].wait()
            scale(i)
            sh[i] = pltpu.async_copy(rows[i % NRB], out_sh.at[dsc5.at[i]],
                                     sem_s[i % NRB], add=True)
            if i + 2 < UNROLL:
                for hnd in ih[i + 2]:
                    hnd.wait()
                coef(i + 2)
                if i >= 1:
                    sh[i - 1].wait()
                gh[i + 2] = pltpu.async_copy(
                    h_hbm.at[src5.at[i + 2]], rows[(i + 2) % NRB],
                    sem_g[(i + 2) % NRB])

        for i in range(UNROLL - NRB, UNROLL):
            sh[i].wait()

    plsc.subcore_barrier()

    wh = []
    for k in range(ZK):
        j = jnp.minimum(s + k * NS, NSEG - 1)
        sl = pl.ds(j * B, B)
        wh.append(
            pltpu.async_copy(out_sh.at[sl], out_hbm.at[c].at[sl], sem_z))
    for hnd in wh:
        hnd.wait()


def _sc_message(h, src3, dst3, w3, dinv):
    mesh = plsc.VectorSubcoreMesh(core_axis_name="c", subcore_axis_name="s")
    return pl.kernel(
        _sc_message_body,
        out_type=jax.ShapeDtypeStruct((NC, N, D), jnp.float32),
        mesh=mesh,
        scratch_types=[
            pltpu.VMEM((UNROLL, B), jnp.int32),
            pltpu.VMEM((UNROLL, B), jnp.int32),
            pltpu.VMEM((UNROLL, B), jnp.int32),
            pltpu.VMEM((UNROLL, B), jnp.float32),
            pltpu.VMEM((UNROLL * B,), jnp.float32),
            pltpu.VMEM((N,), jnp.float32),
            pltpu.VMEM((B, D), jnp.float32),
            pltpu.VMEM((B, D), jnp.float32),
            pltpu.VMEM((B, D), jnp.float32),
            pltpu.SemaphoreType.DMA,
            pltpu.SemaphoreType.DMA,
            pltpu.SemaphoreType.DMA,
            pltpu.SemaphoreType.DMA,
            pltpu.SemaphoreType.DMA,
            pltpu.SemaphoreType.DMA,
            pltpu.SemaphoreType.DMA,
            pltpu.SemaphoreType.DMA,
            pltpu.SemaphoreType.DMA,
            pltpu.SemaphoreType.DMA,
            pltpu.SemaphoreType.DMA,
            pltpu.SemaphoreType.DMA,
            pltpu.VMEM_SHARED((N, D), jnp.float32),
        ],
        compiler_params=_sc_compiler_params(),
    )(h, src3, dst3, w3, dinv)


def _tc_finish_body(p_ref, b_ref, o_ref):
    o_ref[...] = jnp.maximum(p_ref[0] + p_ref[1] + b_ref[...], 0.0)


def _tc_finish(parts, bias2):
    return pl.pallas_call(
        _tc_finish_body,
        out_shape=jax.ShapeDtypeStruct((N, D), jnp.float32),
    )(parts, bias2)


@jax.jit
def kernel(x, edge_index, W, att_src, att_dst, bias):
    att2 = jnp.stack([att_src, att_dst], axis=0)
    alphas = _tc_alphas(x, W, att2)
    h = _tc_h(x, W)
    src3 = edge_index[0].reshape(NW, ROWS_PER_TILE, B)
    dst3 = edge_index[1].reshape(NW, ROWS_PER_TILE, B)
    dpart, w3 = _sc_denom(src3, dst3, alphas[0], alphas[1])
    dinv = _tc_dinv(dpart.reshape(NW, N)).reshape(N)
    parts = _sc_message(h, src3, dst3, w3, dinv)
    return _tc_finish(parts, bias.reshape(1, D))


# speedup vs baseline: 45.0354x; 1.0802x over previous
import dataclasses

import jax
import jax.numpy as jnp
from jax import lax
from jax.experimental import pallas as pl
from jax.experimental.pallas import tpu as pltpu
from jax.experimental.pallas import tpu_sc as plsc

N = 10000
E = 320000
D = 128

NC = 2
NS = 16
NW = NC * NS
EPT = E // NW
B = 80
ROWS_PER_TILE = EPT // B
NSEG = N // B
ZK = (NSEG + NS - 1) // NS


def _sc_compiler_params():
    cp = pltpu.CompilerParams()
    if "needs_layout_passes" in pltpu.CompilerParams.__dataclass_fields__:
        cp = dataclasses.replace(cp, needs_layout_passes=False)
    return cp


def _tc_alphas_body(x_ref, w_ref, att2_ref, al_ref):
    av2 = lax.dot_general(w_ref[...], att2_ref[...], (((1,), (1,)), ((), ())))
    al_ref[...] = lax.dot_general(av2, x_ref[...], (((0,), (1,)), ((), ())))


def _tc_alphas(x, W, att2):
    return pl.pallas_call(
        _tc_alphas_body,
        out_shape=jax.ShapeDtypeStruct((2, N), jnp.float32),
    )(x, W, att2)


def _tc_h_body(x_ref, w_ref, h_ref):
    h_ref[...] = jnp.dot(x_ref[...], w_ref[...])


def _tc_h(x, W):
    return pl.pallas_call(
        _tc_h_body,
        out_shape=jax.ShapeDtypeStruct((N, D), jnp.float32),
    )(x, W)


def _sc_denom_body(src_hbm, dst_hbm, as_hbm, ad_hbm, dpart_hbm, w_hbm,
                   src_v, dst_v, as_v, ad_v, den_v, w_v, sem):
    c = lax.axis_index("c")
    s = lax.axis_index("s")
    wid = s * NC + c

    pltpu.async_copy(src_hbm.at[wid], src_v, sem).wait()
    pltpu.async_copy(dst_hbm.at[wid], dst_v, sem).wait()
    pltpu.async_copy(as_hbm, as_v, sem).wait()
    pltpu.async_copy(ad_hbm, ad_v, sem).wait()

    @pl.loop(0, N // 16)
    def _zero(i):
        den_v[pl.ds(i * 16, 16)] = jnp.zeros((16,), jnp.float32)

    @pl.loop(0, ROWS_PER_TILE)
    def _row(j):
        for i in range(B // 16):
            s16 = src_v[j, pl.ds(i * 16, 16)]
            d16 = dst_v[j, pl.ds(i * 16, 16)]
            a = plsc.load_gather(as_v, [s16])
            b = plsc.load_gather(ad_v, [d16])
            e = a + b
            w = jnp.exp(jnp.maximum(e, 0.2 * e))
            w_v[j, pl.ds(i * 16, 16)] = w
            plsc.addupdate_scatter(den_v, [d16], w)

    pltpu.async_copy(den_v, dpart_hbm.at[pl.ds(wid * N, N)], sem).wait()
    pltpu.async_copy(w_v, w_hbm.at[wid], sem).wait()


def _sc_denom(src3, dst3, as_h, ad_h):
    mesh = plsc.VectorSubcoreMesh(core_axis_name="c", subcore_axis_name="s")
    return pl.kernel(
        _sc_denom_body,
        out_type=[
            jax.ShapeDtypeStruct((NW * N,), jnp.float32),
            jax.ShapeDtypeStruct((NW, ROWS_PER_TILE, B), jnp.float32),
        ],
        mesh=mesh,
        scratch_types=[
            pltpu.VMEM((ROWS_PER_TILE, B), jnp.int32),
            pltpu.VMEM((ROWS_PER_TILE, B), jnp.int32),
            pltpu.VMEM((N,), jnp.float32),
            pltpu.VMEM((N,), jnp.float32),
            pltpu.VMEM((N,), jnp.float32),
            pltpu.VMEM((ROWS_PER_TILE, B), jnp.float32),
            pltpu.SemaphoreType.DMA,
        ],
        compiler_params=_sc_compiler_params(),
    )(src3, dst3, as_h, ad_h)


def _tc_dinv_body(dpart_ref, dinv_ref):
    denom = jnp.sum(dpart_ref[...], axis=0, keepdims=True)
    dinv_ref[...] = 1.0 / (denom + 1e-16)


def _tc_dinv(dpart):
    return pl.pallas_call(
        _tc_dinv_body,
        out_shape=jax.ShapeDtypeStruct((1, N), jnp.float32),
    )(dpart)


UNROLL = 10
NBODY = ROWS_PER_TILE // UNROLL
TAIL = ROWS_PER_TILE - NBODY * UNROLL
NRB = 3


def _sc_message_body(h_hbm, src_hbm, dst_hbm, w_hbm, dinv_hbm, out_hbm,
                     src5, dst5, dsc5, w5, c5, di_v, rows0, rows1, rows2,
                     sem_i0, sem_i1, sem_i2, sem_i3, sem_i4,
                     sem_i5, sem_i6, sem_i7, sem_i8, sem_i9,
                     sem_g0, sem_g1, sem_g2, sem_s0, sem_s1, sem_s2, sem_z,
                     out_sh):
    c = lax.axis_index("c")
    s = lax.axis_index("s")
    wid = s * NC + c

    rows = (rows0, rows1, rows2)
    sem_i = (sem_i0, sem_i1, sem_i2, sem_i3, sem_i4,
             sem_i5, sem_i6, sem_i7, sem_i8, sem_i9)
    sem_g = (sem_g0, sem_g1, sem_g2)
    sem_s = (sem_s0, sem_s1, sem_s2)

    pltpu.async_copy(dinv_hbm, di_v, sem_z).wait()

    @pl.loop(0, B)
    def _zr(r):
        for cc in range(D // 16):
            rows0[r, pl.ds(cc * 16, 16)] = jnp.zeros((16,), jnp.float32)

    zh = []
    for k in range(ZK):
        j = jnp.minimum(s + k * NS, NSEG - 1)
        zh.append(
            pltpu.async_copy(rows0, out_sh.at[pl.ds(j * B, B)], sem_z))
    for hnd in zh:
        hnd.wait()

    plsc.subcore_barrier()

    def coef(i):
        for g in range(B // 16):
            sl = pl.ds(g * 16, 16)
            d16 = dst5[i, 0, sl]
            di = plsc.load_gather(di_v, [d16])
            c5[pl.ds(i * B + g * 16, 16)] = w5[i, 0, sl] * di
            dsc5[i, sl] = d16

    def scale(i):
        @pl.loop(0, B, step=4)
        def _scale(r):
            for rr in range(4):
                cvec = plsc.load_gather(
                    c5, [jnp.full((16,), i * B + r + rr, jnp.int32)]
                )
                for cc in range(D // 16):
                    sl = pl.ds(cc * 16, 16)
                    rows[i % NRB][r + rr, sl] = (
                        rows[i % NRB][r + rr, sl] * cvec)

    def run_chunks(base, unroll):
        row0 = wid * ROWS_PER_TILE + base
        ih = []
        for i in range(unroll):
            ih.append([
                pltpu.async_copy(src_hbm.at[row0 + i], src5.at[i], sem_i[i]),
                pltpu.async_copy(dst_hbm.at[row0 + i], dst5.at[i], sem_i[i]),
                pltpu.async_copy(w_hbm.at[row0 + i], w5.at[i], sem_i[i]),
            ])

        gh = [None] * unroll
        sh = [None] * unroll
        for hnd in ih[0]:
            hnd.wait()
        gh[0] = pltpu.async_copy(h_hbm.at[src5.at[0].at[0]], rows[0],
                                 sem_g[0])
        for hnd in ih[1]:
            hnd.wait()
        gh[1] = pltpu.async_copy(h_hbm.at[src5.at[1].at[0]], rows[1],
                                 sem_g[1])
        coef(0)
        coef(1)

        for i in range(unroll):
            gh[i].wait()
            scale(i)
            sh[i] = pltpu.async_copy(rows[i % NRB], out_sh.at[dsc5.at[i]],
                                     sem_s[i % NRB], add=True)
            if i + 2 < unroll:
                for hnd in ih[i + 2]:
                    hnd.wait()
                coef(i + 2)
                if i >= 1:
                    sh[i - 1].wait()
                gh[i + 2] = pltpu.async_copy(
                    h_hbm.at[src5.at[i + 2].at[0]], rows[(i + 2) % NRB],
                    sem_g[(i + 2) % NRB])

        for i in range(max(0, unroll - NRB), unroll):
            sh[i].wait()

    @pl.loop(0, NBODY)
    def _outer(t):
        run_chunks(t * UNROLL, UNROLL)

    run_chunks(jnp.int32(NBODY * UNROLL), TAIL)

    plsc.subcore_barrier()

    wh = []
    for k in range(ZK):
        j = jnp.minimum(s + k * NS, NSEG - 1)
        sl = pl.ds(j * B, B)
        wh.append(
            pltpu.async_copy(out_sh.at[sl], out_hbm.at[c].at[sl], sem_z))
    for hnd in wh:
        hnd.wait()


def _sc_message(h, src3, dst3, w3, dinv):
    mesh = plsc.VectorSubcoreMesh(core_axis_name="c", subcore_axis_name="s")
    return pl.kernel(
        _sc_message_body,
        out_type=jax.ShapeDtypeStruct((NC, N, D), jnp.float32),
        mesh=mesh,
        scratch_types=[
            pltpu.VMEM((UNROLL, 1, B), jnp.int32),
            pltpu.VMEM((UNROLL, 1, B), jnp.int32),
            pltpu.VMEM((UNROLL, B), jnp.int32),
            pltpu.VMEM((UNROLL, 1, B), jnp.float32),
            pltpu.VMEM((UNROLL * B,), jnp.float32),
            pltpu.VMEM((N,), jnp.float32),
            pltpu.VMEM((B, D), jnp.float32),
            pltpu.VMEM((B, D), jnp.float32),
            pltpu.VMEM((B, D), jnp.float32),
            pltpu.SemaphoreType.DMA,
            pltpu.SemaphoreType.DMA,
            pltpu.SemaphoreType.DMA,
            pltpu.SemaphoreType.DMA,
            pltpu.SemaphoreType.DMA,
            pltpu.SemaphoreType.DMA,
            pltpu.SemaphoreType.DMA,
            pltpu.SemaphoreType.DMA,
            pltpu.SemaphoreType.DMA,
            pltpu.SemaphoreType.DMA,
            pltpu.SemaphoreType.DMA,
            pltpu.SemaphoreType.DMA,
            pltpu.SemaphoreType.DMA,
            pltpu.SemaphoreType.DMA,
            pltpu.SemaphoreType.DMA,
            pltpu.SemaphoreType.DMA,
            pltpu.SemaphoreType.DMA,
            pltpu.VMEM_SHARED((N, D), jnp.float32),
        ],
        compiler_params=_sc_compiler_params(),
    )(h, src3, dst3, w3, dinv)


def _tc_finish_body(p_ref, b_ref, o_ref):
    o_ref[...] = jnp.maximum(p_ref[0] + p_ref[1] + b_ref[...], 0.0)


def _tc_finish(parts, bias2):
    return pl.pallas_call(
        _tc_finish_body,
        out_shape=jax.ShapeDtypeStruct((N, D), jnp.float32),
    )(parts, bias2)


@jax.jit
def kernel(x, edge_index, W, att_src, att_dst, bias):
    att2 = jnp.stack([att_src, att_dst], axis=0)
    alphas = _tc_alphas(x, W, att2)
    h = _tc_h(x, W)
    src3 = edge_index[0].reshape(NW, ROWS_PER_TILE, B)
    dst3 = edge_index[1].reshape(NW, ROWS_PER_TILE, B)
    dpart, w3 = _sc_denom(src3, dst3, alphas[0], alphas[1])
    dinv = _tc_dinv(dpart.reshape(NW, N)).reshape(N)
    src4 = src3.reshape(NW * ROWS_PER_TILE, 1, B)
    dst4 = dst3.reshape(NW * ROWS_PER_TILE, 1, B)
    w4 = w3.reshape(NW * ROWS_PER_TILE, 1, B)
    parts = _sc_message(h, src4, dst4, w4, dinv)
    return _tc_finish(parts, bias.reshape(1, D))


# speedup vs baseline: 46.2040x; 1.0259x over previous
import dataclasses

import jax
import jax.numpy as jnp
from jax import lax
from jax.experimental import pallas as pl
from jax.experimental.pallas import tpu as pltpu
from jax.experimental.pallas import tpu_sc as plsc

N = 10000
E = 320000
D = 128

NC = 2
NS = 16
NW = NC * NS
EPT = E // NW
B = 80
ROWS_PER_TILE = EPT // B
NSEG = N // B
ZK = (NSEG + NS - 1) // NS


def _sc_compiler_params():
    cp = pltpu.CompilerParams()
    if "needs_layout_passes" in pltpu.CompilerParams.__dataclass_fields__:
        cp = dataclasses.replace(cp, needs_layout_passes=False)
    return cp


def _tc_alphas_body(x_ref, w_ref, att2_ref, al_ref):
    av2 = lax.dot_general(w_ref[...], att2_ref[...], (((1,), (1,)), ((), ())))
    al_ref[...] = lax.dot_general(av2, x_ref[...], (((0,), (1,)), ((), ())))


def _tc_alphas(x, W, att2):
    return pl.pallas_call(
        _tc_alphas_body,
        out_shape=jax.ShapeDtypeStruct((2, N), jnp.float32),
    )(x, W, att2)


def _tc_h_body(x_ref, w_ref, h_ref):
    h_ref[...] = jnp.dot(x_ref[...], w_ref[...])


def _tc_h(x, W):
    return pl.pallas_call(
        _tc_h_body,
        out_shape=jax.ShapeDtypeStruct((N, D), jnp.float32),
    )(x, W)


def _sc_denom_body(src_hbm, dst_hbm, as_hbm, ad_hbm, dpart_hbm, w_hbm,
                   src_v, dst_v, as_v, ad_v, den_v, w_v, sem):
    c = lax.axis_index("c")
    s = lax.axis_index("s")
    wid = s * NC + c

    pltpu.async_copy(src_hbm.at[wid], src_v, sem).wait()
    pltpu.async_copy(dst_hbm.at[wid], dst_v, sem).wait()
    pltpu.async_copy(as_hbm, as_v, sem).wait()
    pltpu.async_copy(ad_hbm, ad_v, sem).wait()

    @pl.loop(0, N // 16)
    def _zero(i):
        den_v[pl.ds(i * 16, 16)] = jnp.zeros((16,), jnp.float32)

    @pl.loop(0, ROWS_PER_TILE)
    def _row(j):
        for i in range(B // 16):
            s16 = src_v[j, pl.ds(i * 16, 16)]
            d16 = dst_v[j, pl.ds(i * 16, 16)]
            a = plsc.load_gather(as_v, [s16])
            b = plsc.load_gather(ad_v, [d16])
            e = a + b
            w = jnp.exp(jnp.maximum(e, 0.2 * e))
            w_v[j, pl.ds(i * 16, 16)] = w
            plsc.addupdate_scatter(den_v, [d16], w)

    pltpu.async_copy(den_v, dpart_hbm.at[pl.ds(wid * N, N)], sem).wait()
    pltpu.async_copy(w_v, w_hbm.at[wid], sem).wait()


def _sc_denom(src3, dst3, as_h, ad_h):
    mesh = plsc.VectorSubcoreMesh(core_axis_name="c", subcore_axis_name="s")
    return pl.kernel(
        _sc_denom_body,
        out_type=[
            jax.ShapeDtypeStruct((NW * N,), jnp.float32),
            jax.ShapeDtypeStruct((NW, ROWS_PER_TILE, B), jnp.float32),
        ],
        mesh=mesh,
        scratch_types=[
            pltpu.VMEM((ROWS_PER_TILE, B), jnp.int32),
            pltpu.VMEM((ROWS_PER_TILE, B), jnp.int32),
            pltpu.VMEM((N,), jnp.float32),
            pltpu.VMEM((N,), jnp.float32),
            pltpu.VMEM((N,), jnp.float32),
            pltpu.VMEM((ROWS_PER_TILE, B), jnp.float32),
            pltpu.SemaphoreType.DMA,
        ],
        compiler_params=_sc_compiler_params(),
    )(src3, dst3, as_h, ad_h)


UNROLL = 10
NBODY = ROWS_PER_TILE // UNROLL
TAIL = ROWS_PER_TILE - NBODY * UNROLL
NRB = 3


def _sc_message_body(h_hbm, src_hbm, dst_hbm, w_hbm, out_hbm,
                     src5, dst5, w5, c5, rows0, rows1, rows2,
                     sem_i0, sem_i1, sem_i2, sem_i3, sem_i4,
                     sem_i5, sem_i6, sem_i7, sem_i8, sem_i9,
                     sem_g0, sem_g1, sem_g2, sem_s0, sem_s1, sem_s2, sem_z,
                     out_sh):
    c = lax.axis_index("c")
    s = lax.axis_index("s")
    wid = s * NC + c

    rows = (rows0, rows1, rows2)
    sem_i = (sem_i0, sem_i1, sem_i2, sem_i3, sem_i4,
             sem_i5, sem_i6, sem_i7, sem_i8, sem_i9)
    sem_g = (sem_g0, sem_g1, sem_g2)
    sem_s = (sem_s0, sem_s1, sem_s2)

    @pl.loop(0, B)
    def _zr(r):
        for cc in range(D // 16):
            rows0[r, pl.ds(cc * 16, 16)] = jnp.zeros((16,), jnp.float32)

    zh = []
    for k in range(ZK):
        j = jnp.minimum(s + k * NS, NSEG - 1)
        zh.append(
            pltpu.async_copy(rows0, out_sh.at[pl.ds(j * B, B)], sem_z))
    for hnd in zh:
        hnd.wait()

    plsc.subcore_barrier()

    def coef(i):
        for g in range(B // 16):
            c5[pl.ds(i * B + g * 16, 16)] = w5[i, 0, pl.ds(g * 16, 16)]

    def scale(i):
        @pl.loop(0, B, step=4)
        def _scale(r):
            for rr in range(4):
                cvec = plsc.load_gather(
                    c5, [jnp.full((16,), i * B + r + rr, jnp.int32)]
                )
                for cc in range(D // 16):
                    sl = pl.ds(cc * 16, 16)
                    rows[i % NRB][r + rr, sl] = (
                        rows[i % NRB][r + rr, sl] * cvec)

    def run_chunks(base, unroll):
        row0 = wid * ROWS_PER_TILE + base
        ih = []
        for i in range(unroll):
            ih.append([
                pltpu.async_copy(src_hbm.at[row0 + i], src5.at[i], sem_i[i]),
                pltpu.async_copy(dst_hbm.at[row0 + i], dst5.at[i], sem_i[i]),
                pltpu.async_copy(w_hbm.at[row0 + i], w5.at[i], sem_i[i]),
            ])

        gh = [None] * unroll
        sh = [None] * unroll
        for hnd in ih[0]:
            hnd.wait()
        gh[0] = pltpu.async_copy(h_hbm.at[src5.at[0].at[0]], rows[0],
                                 sem_g[0])
        for hnd in ih[1]:
            hnd.wait()
        gh[1] = pltpu.async_copy(h_hbm.at[src5.at[1].at[0]], rows[1],
                                 sem_g[1])
        coef(0)
        coef(1)

        for i in range(unroll):
            gh[i].wait()
            scale(i)
            sh[i] = pltpu.async_copy(rows[i % NRB],
                                     out_sh.at[dst5.at[i].at[0]],
                                     sem_s[i % NRB], add=True)
            if i + 2 < unroll:
                for hnd in ih[i + 2]:
                    hnd.wait()
                coef(i + 2)
                if i >= 1:
                    sh[i - 1].wait()
                gh[i + 2] = pltpu.async_copy(
                    h_hbm.at[src5.at[i + 2].at[0]], rows[(i + 2) % NRB],
                    sem_g[(i + 2) % NRB])

        for i in range(max(0, unroll - NRB), unroll):
            sh[i].wait()

    @pl.loop(0, NBODY)
    def _outer(t):
        run_chunks(t * UNROLL, UNROLL)

    run_chunks(jnp.int32(NBODY * UNROLL), TAIL)

    plsc.subcore_barrier()

    wh = []
    for k in range(ZK):
        j = jnp.minimum(s + k * NS, NSEG - 1)
        sl = pl.ds(j * B, B)
        wh.append(
            pltpu.async_copy(out_sh.at[sl], out_hbm.at[c].at[sl], sem_z))
    for hnd in wh:
        hnd.wait()


def _sc_message(h, src4, dst4, w4):
    mesh = plsc.VectorSubcoreMesh(core_axis_name="c", subcore_axis_name="s")
    return pl.kernel(
        _sc_message_body,
        out_type=jax.ShapeDtypeStruct((NC, N, D), jnp.float32),
        mesh=mesh,
        scratch_types=[
            pltpu.VMEM((UNROLL, 1, B), jnp.int32),
            pltpu.VMEM((UNROLL, 1, B), jnp.int32),
            pltpu.VMEM((UNROLL, 1, B), jnp.float32),
            pltpu.VMEM((UNROLL * B,), jnp.float32),
            pltpu.VMEM((B, D), jnp.float32),
            pltpu.VMEM((B, D), jnp.float32),
            pltpu.VMEM((B, D), jnp.float32),
            pltpu.SemaphoreType.DMA,
            pltpu.SemaphoreType.DMA,
            pltpu.SemaphoreType.DMA,
            pltpu.SemaphoreType.DMA,
            pltpu.SemaphoreType.DMA,
            pltpu.SemaphoreType.DMA,
            pltpu.SemaphoreType.DMA,
            pltpu.SemaphoreType.DMA,
            pltpu.SemaphoreType.DMA,
            pltpu.SemaphoreType.DMA,
            pltpu.SemaphoreType.DMA,
            pltpu.SemaphoreType.DMA,
            pltpu.SemaphoreType.DMA,
            pltpu.SemaphoreType.DMA,
            pltpu.SemaphoreType.DMA,
            pltpu.SemaphoreType.DMA,
            pltpu.SemaphoreType.DMA,
            pltpu.VMEM_SHARED((N, D), jnp.float32),
        ],
        compiler_params=_sc_compiler_params(),
    )(h, src4, dst4, w4)


def _tc_finish_body(p_ref, dpart_ref, b_ref, o_ref):
    ones = jnp.ones((NW, 1), jnp.float32)
    den_col = lax.dot_general(dpart_ref[...], ones, (((0,), (0,)), ((), ())))
    dinv = 1.0 / (den_col + 1e-16)
    o_ref[...] = jnp.maximum(
        (p_ref[0] + p_ref[1]) * dinv + b_ref[...], 0.0)


def _tc_finish(parts, dpart, bias2):
    return pl.pallas_call(
        _tc_finish_body,
        out_shape=jax.ShapeDtypeStruct((N, D), jnp.float32),
    )(parts, dpart, bias2)


@jax.jit
def kernel(x, edge_index, W, att_src, att_dst, bias):
    att2 = jnp.stack([att_src, att_dst], axis=0)
    alphas = _tc_alphas(x, W, att2)
    h = _tc_h(x, W)
    src3 = edge_index[0].reshape(NW, ROWS_PER_TILE, B)
    dst3 = edge_index[1].reshape(NW, ROWS_PER_TILE, B)
    dpart, w3 = _sc_denom(src3, dst3, alphas[0], alphas[1])
    src4 = src3.reshape(NW * ROWS_PER_TILE, 1, B)
    dst4 = dst3.reshape(NW * ROWS_PER_TILE, 1, B)
    w4 = w3.reshape(NW * ROWS_PER_TILE, 1, B)
    parts = _sc_message(h, src4, dst4, w4)
    return _tc_finish(parts, dpart.reshape(NW, N), bias.reshape(1, D))


# speedup vs baseline: 48.5832x; 1.0515x over previous
import dataclasses

import jax
import jax.numpy as jnp
from jax import lax
from jax.experimental import pallas as pl
from jax.experimental.pallas import tpu as pltpu
from jax.experimental.pallas import tpu_sc as plsc

N = 10000
E = 320000
D = 128

NC = 2
NS = 16
NW = NC * NS
EPT = E // NW
B = 80
ROWS_PER_TILE = EPT // B
NSEG = N // B
ZK = (NSEG + NS - 1) // NS


def _sc_compiler_params():
    cp = pltpu.CompilerParams()
    if "needs_layout_passes" in pltpu.CompilerParams.__dataclass_fields__:
        cp = dataclasses.replace(cp, needs_layout_passes=False)
    return cp


def _tc_alphas_body(x_ref, w_ref, att2_ref, al_ref):
    av2 = lax.dot_general(w_ref[...], att2_ref[...], (((1,), (1,)), ((), ())))
    al_ref[...] = lax.dot_general(av2, x_ref[...], (((0,), (1,)), ((), ())))


def _tc_alphas(x, W, att2):
    return pl.pallas_call(
        _tc_alphas_body,
        out_shape=jax.ShapeDtypeStruct((2, N), jnp.float32),
    )(x, W, att2)


def _tc_h_body(x_ref, w_ref, h_ref):
    h_ref[...] = jnp.dot(x_ref[...], w_ref[...])


def _tc_h(x, W):
    return pl.pallas_call(
        _tc_h_body,
        out_shape=jax.ShapeDtypeStruct((N, D), jnp.float32),
    )(x, W)


def _sc_denom_body(src_hbm, dst_hbm, as_hbm, ad_hbm, dpart_hbm, w_hbm,
                   src_v, dst_v, as_v, ad_v, den_v, w_v, sem):
    c = lax.axis_index("c")
    s = lax.axis_index("s")
    wid = s * NC + c

    pltpu.async_copy(src_hbm.at[wid], src_v, sem).wait()
    pltpu.async_copy(dst_hbm.at[wid], dst_v, sem).wait()
    pltpu.async_copy(as_hbm, as_v, sem).wait()
    pltpu.async_copy(ad_hbm, ad_v, sem).wait()

    @pl.loop(0, N // 16)
    def _zero(i):
        den_v[pl.ds(i * 16, 16)] = jnp.zeros((16,), jnp.float32)

    @pl.loop(0, ROWS_PER_TILE)
    def _row(j):
        for i in range(B // 16):
            s16 = src_v[j, pl.ds(i * 16, 16)]
            d16 = dst_v[j, pl.ds(i * 16, 16)]
            a = plsc.load_gather(as_v, [s16])
            b = plsc.load_gather(ad_v, [d16])
            e = a + b
            w = jnp.exp(jnp.maximum(e, 0.2 * e))
            w_v[j, pl.ds(i * 16, 16)] = w
            plsc.addupdate_scatter(den_v, [d16], w)

    pltpu.async_copy(den_v, dpart_hbm.at[pl.ds(wid * N, N)], sem).wait()
    pltpu.async_copy(w_v, w_hbm.at[wid], sem).wait()


def _sc_denom(src3, dst3, as_h, ad_h):
    mesh = plsc.VectorSubcoreMesh(core_axis_name="c", subcore_axis_name="s")
    return pl.kernel(
        _sc_denom_body,
        out_type=[
            jax.ShapeDtypeStruct((NW * N,), jnp.float32),
            jax.ShapeDtypeStruct((NW, ROWS_PER_TILE, B), jnp.float32),
        ],
        mesh=mesh,
        scratch_types=[
            pltpu.VMEM((ROWS_PER_TILE, B), jnp.int32),
            pltpu.VMEM((ROWS_PER_TILE, B), jnp.int32),
            pltpu.VMEM((N,), jnp.float32),
            pltpu.VMEM((N,), jnp.float32),
            pltpu.VMEM((N,), jnp.float32),
            pltpu.VMEM((ROWS_PER_TILE, B), jnp.float32),
            pltpu.SemaphoreType.DMA,
        ],
        compiler_params=_sc_compiler_params(),
    )(src3, dst3, as_h, ad_h)


UNROLL = 25
NBODY = ROWS_PER_TILE // UNROLL
NRB = 3
NIW = 5


def _sc_message_body(h_hbm, src_hbm, dst_hbm, w_hbm, out_hbm,
                     src5, dst5, w5, dsc3, c3, rows0, rows1, rows2,
                     sem_i0, sem_i1, sem_i2, sem_i3, sem_i4,
                     sem_g0, sem_g1, sem_g2, sem_s0, sem_s1, sem_s2, sem_z,
                     out_sh):
    c = lax.axis_index("c")
    s = lax.axis_index("s")
    wid = s * NC + c

    rows = (rows0, rows1, rows2)
    sem_i = (sem_i0, sem_i1, sem_i2, sem_i3, sem_i4)
    sem_g = (sem_g0, sem_g1, sem_g2)
    sem_s = (sem_s0, sem_s1, sem_s2)

    @pl.loop(0, B)
    def _zr(r):
        for cc in range(D // 16):
            rows0[r, pl.ds(cc * 16, 16)] = jnp.zeros((16,), jnp.float32)

    zh = []
    for k in range(ZK):
        j = jnp.minimum(s + k * NS, NSEG - 1)
        zh.append(
            pltpu.async_copy(rows0, out_sh.at[pl.ds(j * B, B)], sem_z))
    for hnd in zh:
        hnd.wait()

    plsc.subcore_barrier()

    def coef(i):
        for g in range(B // 16):
            sl = pl.ds(g * 16, 16)
            c3[pl.ds((i % NRB) * B + g * 16, 16)] = w5[i % NIW, 0, sl]
            dsc3[i % NRB, sl] = dst5[i % NIW, 0, sl]

    def scale(i):
        @pl.loop(0, B, step=4)
        def _scale(r):
            for rr in range(4):
                cvec = plsc.load_gather(
                    c3, [jnp.full((16,), (i % NRB) * B + r + rr, jnp.int32)]
                )
                for cc in range(D // 16):
                    sl = pl.ds(cc * 16, 16)
                    rows[i % NRB][r + rr, sl] = (
                        rows[i % NRB][r + rr, sl] * cvec)

    def run_chunks(base, unroll):
        row0 = wid * ROWS_PER_TILE + base
        ih = [None] * unroll

        def issue(i):
            ih[i] = [
                pltpu.async_copy(src_hbm.at[row0 + i], src5.at[i % NIW],
                                 sem_i[i % NIW]),
                pltpu.async_copy(dst_hbm.at[row0 + i], dst5.at[i % NIW],
                                 sem_i[i % NIW]),
                pltpu.async_copy(w_hbm.at[row0 + i], w5.at[i % NIW],
                                 sem_i[i % NIW]),
            ]

        def gather(i):
            return pltpu.async_copy(
                h_hbm.at[src5.at[i % NIW].at[0]], rows[i % NRB],
                sem_g[i % NRB])

        for i in range(min(NIW, unroll)):
            issue(i)

        gh = [None] * unroll
        sh = [None] * unroll
        for hnd in ih[0]:
            hnd.wait()
        gh[0] = gather(0)
        for hnd in ih[1]:
            hnd.wait()
        gh[1] = gather(1)
        coef(0)
        coef(1)

        for i in range(unroll):
            gh[i].wait()
            scale(i)
            sh[i] = pltpu.async_copy(rows[i % NRB], out_sh.at[dsc3.at[i % NRB]],
                                     sem_s[i % NRB], add=True)
            if i + 2 < unroll:
                for hnd in ih[i + 2]:
                    hnd.wait()
                if i >= 1:
                    sh[i - 1].wait()
                coef(i + 2)
                gh[i + 2] = gather(i + 2)
                if i + NIW < unroll:
                    issue(i + NIW)

        for i in range(max(0, unroll - NRB), unroll):
            sh[i].wait()

    @pl.loop(0, NBODY)
    def _outer(t):
        run_chunks(t * UNROLL, UNROLL)

    plsc.subcore_barrier()

    wh = []
    for k in range(ZK):
        j = jnp.minimum(s + k * NS, NSEG - 1)
        sl = pl.ds(j * B, B)
        wh.append(
            pltpu.async_copy(out_sh.at[sl], out_hbm.at[c].at[sl], sem_z))
    for hnd in wh:
        hnd.wait()


def _sc_message(h, src4, dst4, w4):
    mesh = plsc.VectorSubcoreMesh(core_axis_name="c", subcore_axis_name="s")
    return pl.kernel(
        _sc_message_body,
        out_type=jax.ShapeDtypeStruct((NC, N, D), jnp.float32),
        mesh=mesh,
        scratch_types=[
            pltpu.VMEM((NIW, 1, B), jnp.int32),
            pltpu.VMEM((NIW, 1, B), jnp.int32),
            pltpu.VMEM((NIW, 1, B), jnp.float32),
            pltpu.VMEM((NRB, B), jnp.int32),
            pltpu.VMEM((NRB * B,), jnp.float32),
            pltpu.VMEM((B, D), jnp.float32),
            pltpu.VMEM((B, D), jnp.float32),
            pltpu.VMEM((B, D), jnp.float32),
            pltpu.SemaphoreType.DMA,
            pltpu.SemaphoreType.DMA,
            pltpu.SemaphoreType.DMA,
            pltpu.SemaphoreType.DMA,
            pltpu.SemaphoreType.DMA,
            pltpu.SemaphoreType.DMA,
            pltpu.SemaphoreType.DMA,
            pltpu.SemaphoreType.DMA,
            pltpu.SemaphoreType.DMA,
            pltpu.SemaphoreType.DMA,
            pltpu.SemaphoreType.DMA,
            pltpu.SemaphoreType.DMA,
            pltpu.VMEM_SHARED((N, D), jnp.float32),
        ],
        compiler_params=_sc_compiler_params(),
    )(h, src4, dst4, w4)


def _tc_finish_body(p_ref, dpart_ref, b_ref, o_ref):
    ones = jnp.ones((NW, 1), jnp.float32)
    den_col = lax.dot_general(dpart_ref[...], ones, (((0,), (0,)), ((), ())))
    dinv = 1.0 / (den_col + 1e-16)
    o_ref[...] = jnp.maximum(
        (p_ref[0] + p_ref[1]) * dinv + b_ref[...], 0.0)


def _tc_finish(parts, dpart, bias2):
    return pl.pallas_call(
        _tc_finish_body,
        out_shape=jax.ShapeDtypeStruct((N, D), jnp.float32),
    )(parts, dpart, bias2)


@jax.jit
def kernel(x, edge_index, W, att_src, att_dst, bias):
    att2 = jnp.stack([att_src, att_dst], axis=0)
    alphas = _tc_alphas(x, W, att2)
    h = _tc_h(x, W)
    src3 = edge_index[0].reshape(NW, ROWS_PER_TILE, B)
    dst3 = edge_index[1].reshape(NW, ROWS_PER_TILE, B)
    dpart, w3 = _sc_denom(src3, dst3, alphas[0], alphas[1])
    src4 = src3.reshape(NW * ROWS_PER_TILE, 1, B)
    dst4 = dst3.reshape(NW * ROWS_PER_TILE, 1, B)
    w4 = w3.reshape(NW * ROWS_PER_TILE, 1, B)
    parts = _sc_message(h, src4, dst4, w4)
    return _tc_finish(parts, dpart.reshape(NW, N), bias.reshape(1, D))
